# baseline (speedup 1.0000x reference)
import jax
import jax.numpy as jnp
from jax.experimental import pallas as pl
from jax.experimental.pallas import tpu as pltpu

B, A, D, KM1, K_TOP = 64, 4096, 256, 64, 64
B_TILE = 8
A_TILE = 2048


def _body(x_ref, w_ref, b_ref, o_ref, sc_ref):
    j = pl.program_id(1)
    x = x_ref[...].reshape(B_TILE * A_TILE, D)
    logits_t = jax.lax.dot_general(
        w_ref[...], x, (((0,), (1,)), ((), ())),
        preferred_element_type=jnp.float32)
    s = jax.nn.sigmoid(logits_t + b_ref[...])
    sc_ref[j] = (s.sum(axis=0) * (1.0 / KM1) - 0.5).reshape(B_TILE, A_TILE)

    @pl.when(j == (A // A_TILE) - 1)
    def _select():
        st = jnp.concatenate([sc_ref[k] for k in range(A // A_TILE)], axis=1)
        at = jnp.abs(st)
        bits = jax.lax.bitcast_convert_type(at, jnp.int32)
        iota = jax.lax.broadcasted_iota(jnp.int32, (B_TILE, A), 1)

        def vstep(_, c):
            lo, hi = c
            mid = lo + ((hi - lo) >> 1)
            cnt = jnp.sum((bits >= mid).astype(jnp.int32), axis=1,
                          keepdims=True)
            ge = cnt >= K_TOP
            return jnp.where(ge, mid, lo), jnp.where(ge, hi, mid)

        t, _ = jax.lax.fori_loop(
            0, 31, vstep,
            (jnp.zeros((B_TILE, 1), jnp.int32),
             jnp.full((B_TILE, 1), 0x7F800000, jnp.int32)))

        gt = bits > t
        eq = bits == t
        n_gt = jnp.sum(gt.astype(jnp.int32), axis=1, keepdims=True)

        def istep(_, c):
            lo, hi = c
            mid = lo + ((hi - lo + 1) >> 1)
            cnt = n_gt + jnp.sum((eq & (iota <= mid)).astype(jnp.int32),
                                 axis=1, keepdims=True)
            ge = cnt >= K_TOP
            return jnp.where(ge, lo, mid), jnp.where(ge, mid, hi)

        _, i_thr = jax.lax.fori_loop(
            0, 12, istep,
            (jnp.full((B_TILE, 1), -1, jnp.int32),
             jnp.full((B_TILE, 1), A - 1, jnp.int32)))

        mask = gt | (eq & (iota <= i_thr))
        sel = jnp.where(mask, st, 0.0)
        z = jnp.sum(jnp.abs(sel), axis=1, keepdims=True)
        o_ref[...] = sel / (z + 1e-8)


@jax.jit
def kernel(signal_features, W, b):
    return pl.pallas_call(
        _body,
        grid=(B // B_TILE, A // A_TILE),
        in_specs=[
            pl.BlockSpec((B_TILE, A_TILE, D), lambda i, j: (i, j, 0)),
            pl.BlockSpec((D, KM1), lambda i, j: (0, 0)),
            pl.BlockSpec((KM1, 1), lambda i, j: (0, 0)),
        ],
        out_specs=pl.BlockSpec((B_TILE, A), lambda i, j: (i, 0)),
        out_shape=jax.ShapeDtypeStruct((B, A), jnp.float32),
        scratch_shapes=[pltpu.VMEM((A // A_TILE, B_TILE, A_TILE), jnp.float32)],
    )(signal_features, W, b.reshape(KM1, 1))
